# Initial kernel scaffold; baseline (speedup 1.0000x reference)
import jax
import jax.numpy as jnp
from jax.experimental import pallas as pl


def kernel(pos_u, pos_v, neg_v, u_weight, v_weight):
    raise NotImplementedError("write your pallas kernel here")


# speedup vs baseline: 1.5828x; 1.5828x over previous
import functools

import jax
import jax.numpy as jnp
from jax import lax
from jax.experimental import pallas as pl
from jax.experimental.pallas import tpu as pltpu
from jax.experimental.pallas import tpu_sc as plsc

NUM_NODES = 1000000
DIM = 64
BATCH = 16384
NEG = 5

NC = 2
NS = 16
L = 16
NW = NC * NS
BPW = BATCH // NW
CHUNK = 128
NCHUNK = BPW // CHUNK
NGROUP = CHUNK // L


def _sc_scores(pos_u, pos_v, neg_flat, u_weight, v_weight):
    mesh = plsc.VectorSubcoreMesh(core_axis_name="c", subcore_axis_name="s")

    @functools.partial(
        pl.kernel,
        mesh=mesh,
        compiler_params=pltpu.CompilerParams(
            use_tc_tiling_on_sc=False, needs_layout_passes=False),
        out_type=[
            jax.ShapeDtypeStruct((BATCH,), jnp.float32),
            jax.ShapeDtypeStruct((NEG * BATCH,), jnp.float32),
        ],
        scratch_types=[
            pltpu.VMEM((CHUNK,), jnp.int32),
            pltpu.VMEM((CHUNK,), jnp.int32),
            pltpu.VMEM((CHUNK * NEG,), jnp.int32),
            pltpu.VMEM((CHUNK, DIM), jnp.float32),
            pltpu.VMEM((CHUNK, DIM), jnp.float32),
            pltpu.VMEM((CHUNK * NEG, DIM), jnp.float32),
            pltpu.VMEM((BPW,), jnp.float32),
            pltpu.VMEM((NEG * BPW,), jnp.float32),
            pltpu.SemaphoreType.DMA,
            pltpu.SemaphoreType.DMA,
            pltpu.SemaphoreType.DMA,
        ],
    )
    def body(pos_u_hbm, pos_v_hbm, neg_hbm, uw_hbm, vw_hbm,
             out_pos_hbm, out_neg_hbm,
             idx_u, idx_v, idx_n, u_rows, v_rows, n_rows,
             pos_sc, neg_sc, sem_u, sem_v, sem_n):
        wid = lax.axis_index("s") * NC + lax.axis_index("c")
        wbase = pl.multiple_of(wid * BPW, BPW)
        lanes = lax.broadcasted_iota(jnp.int32, (L,), 0)

        for ci in range(NCHUNK):
            base = pl.multiple_of(wbase + ci * CHUNK, CHUNK)
            nbase = pl.multiple_of(base * NEG, CHUNK * NEG)
            pltpu.sync_copy(pos_u_hbm.at[pl.ds(base, CHUNK)], idx_u)
            pltpu.sync_copy(pos_v_hbm.at[pl.ds(base, CHUNK)], idx_v)
            pltpu.sync_copy(neg_hbm.at[pl.ds(nbase, CHUNK * NEG)], idx_n)
            cp_u = pltpu.async_copy(uw_hbm.at[idx_u], u_rows, sem_u)
            cp_v = pltpu.async_copy(vw_hbm.at[idx_v], v_rows, sem_v)
            cp_n = pltpu.async_copy(vw_hbm.at[idx_n], n_rows, sem_n)
            cp_u.wait()
            cp_v.wait()
            cp_n.wait()

            def group_body(g, _, ci=ci):
                b16 = g * L + lanes
                acc0 = jnp.zeros((L,), jnp.float32)

                def dot_body(d, carry):
                    p, n0, n1, n2, n3, n4 = carry
                    dv = jnp.full((L,), d, jnp.int32)
                    u_d = plsc.load_gather(u_rows, [b16, dv])
                    v_d = plsc.load_gather(v_rows, [b16, dv])
                    p = p + u_d * v_d
                    bn = b16 * NEG
                    g0 = plsc.load_gather(n_rows, [bn, dv])
                    g1 = plsc.load_gather(n_rows, [bn + 1, dv])
                    g2 = plsc.load_gather(n_rows, [bn + 2, dv])
                    g3 = plsc.load_gather(n_rows, [bn + 3, dv])
                    g4 = plsc.load_gather(n_rows, [bn + 4, dv])
                    return (p, n0 + g0 * u_d, n1 + g1 * u_d, n2 + g2 * u_d,
                            n3 + g3 * u_d, n4 + g4 * u_d)

                p, n0, n1, n2, n3, n4 = lax.fori_loop(
                    0, DIM, dot_body, (acc0, acc0, acc0, acc0, acc0, acc0))
                off = ci * CHUNK + g * L
                pos_sc[pl.ds(off, L)] = p
                neg_sc[pl.ds(off, L)] = n0
                neg_sc[pl.ds(BPW + off, L)] = n1
                neg_sc[pl.ds(2 * BPW + off, L)] = n2
                neg_sc[pl.ds(3 * BPW + off, L)] = n3
                neg_sc[pl.ds(4 * BPW + off, L)] = n4
                return 0

            lax.fori_loop(0, NGROUP, group_body, 0)

        pltpu.sync_copy(pos_sc, out_pos_hbm.at[pl.ds(wbase, BPW)])
        for n in range(NEG):
            pltpu.sync_copy(
                neg_sc.at[pl.ds(n * BPW, BPW)],
                out_neg_hbm.at[pl.ds(pl.multiple_of(n * BATCH + wbase, BPW), BPW)])

    return body(pos_u, pos_v, neg_flat, u_weight, v_weight)


def _tc_loss(pos_scores, neg_scores):
    def body(p_ref, n_ref, out_ref):
        p = jnp.clip(p_ref[...], -10.0, 10.0)
        n = jnp.clip(n_ref[...], -10.0, 10.0)
        lp = jnp.maximum(-p, 0.0) + jnp.log1p(jnp.exp(-jnp.abs(p)))
        ln = jnp.maximum(n, 0.0) + jnp.log1p(jnp.exp(-jnp.abs(n)))
        total = (jnp.sum(lp) + jnp.sum(ln)) * (1.0 / BATCH)
        out_ref[...] = total[None, None]

    out = pl.pallas_call(
        body,
        out_shape=jax.ShapeDtypeStruct((1, 1), jnp.float32),
    )(pos_scores, neg_scores)
    return out[0, 0]


def kernel(pos_u, pos_v, neg_v, u_weight, v_weight):
    pos_u = pos_u.astype(jnp.int32)
    pos_v = pos_v.astype(jnp.int32)
    neg_flat = neg_v.astype(jnp.int32).reshape(BATCH * NEG)
    pos_scores, neg_scores = _sc_scores(pos_u, pos_v, neg_flat,
                                        u_weight, v_weight)
    return _tc_loss(pos_scores.reshape(128, 128),
                    neg_scores.reshape(NEG * 128, 128))
